# Initial kernel scaffold; baseline (speedup 1.0000x reference)
import jax
import jax.numpy as jnp
from jax.experimental import pallas as pl


def kernel(inputs, target_data):
    raise NotImplementedError("write your pallas kernel here")


# speedup vs baseline: 1.4214x; 1.4214x over previous
import functools

import jax
import jax.numpy as jnp
from jax import lax
from jax.experimental import pallas as pl
from jax.experimental.pallas import tpu as pltpu
from jax.experimental.pallas import tpu_sc as plsc

Q = 1024
D = 128
K = 100000
NN = 32

BK = 2048
NKT = (K + BK - 1) // BK
K_PAD = NKT * BK

_BIG = 3.0e38
_BIGI = 2**31 - 1


def _topk_kernel(q_ref, tt_ref, oidx_ref, rv_ref, ri_ref):
    kt = pl.program_id(0)

    @pl.when(kt == 0)
    def _init():
        rv_ref[...] = jnp.full((Q, NN), _BIG, jnp.float32)
        ri_ref[...] = jnp.full((Q, NN), _BIGI, jnp.int32)

    qm = q_ref[...]
    tt = tt_ref[...]
    t2 = jnp.sum(tt * tt, axis=0, keepdims=True)
    s = t2 - 2.0 * jnp.dot(qm, tt, preferred_element_type=jnp.float32)

    col = lax.broadcasted_iota(jnp.int32, (Q, BK), 1)
    gcol = col + kt * BK
    s = jnp.where(gcol >= K, _BIG, s)

    lv = jnp.full((Q, NN), _BIG, jnp.float32)
    li = jnp.full((Q, NN), _BIGI, jnp.int32)
    j32 = lax.broadcasted_iota(jnp.int32, (Q, NN), 1)
    for j in range(NN):
        m = jnp.min(s, axis=1, keepdims=True)
        hit = s == m
        sel = jnp.min(jnp.where(hit, col, _BIGI), axis=1, keepdims=True)
        lv = jnp.where(j32 == j, m, lv)
        li = jnp.where(j32 == j, sel + kt * BK, li)
        s = jnp.where(col == sel, _BIG, s)

    rv = rv_ref[...]
    ri = ri_ref[...]
    nv = jnp.full((Q, NN), _BIG, jnp.float32)
    ni = jnp.full((Q, NN), _BIGI, jnp.int32)
    for j in range(NN):
        ma = jnp.min(rv, axis=1, keepdims=True)
        mb = jnp.min(lv, axis=1, keepdims=True)
        ia = jnp.min(jnp.where(rv == ma, ri, _BIGI), axis=1, keepdims=True)
        ib = jnp.min(jnp.where(lv == mb, li, _BIGI), axis=1, keepdims=True)
        take_a = (ma < mb) | ((ma == mb) & (ia < ib))
        val = jnp.where(take_a, ma, mb)
        idx = jnp.where(take_a, ia, ib)
        nv = jnp.where(j32 == j, val, nv)
        ni = jnp.where(j32 == j, idx, ni)
        rv = jnp.where(take_a & (ri == ia), _BIG, rv)
        lv = jnp.where((~take_a) & (li == ib), _BIG, lv)
    rv_ref[...] = nv
    ri_ref[...] = ni

    @pl.when(kt == NKT - 1)
    def _out():
        oidx_ref[...] = ni


def _topk_indices(inputs, tt_pad):
    return pl.pallas_call(
        _topk_kernel,
        grid=(NKT,),
        in_specs=[
            pl.BlockSpec((Q, D), lambda k: (0, 0)),
            pl.BlockSpec((D, BK), lambda k: (0, k)),
        ],
        out_specs=pl.BlockSpec((Q, NN), lambda k: (0, 0)),
        out_shape=jax.ShapeDtypeStruct((Q, NN), jnp.int32),
        scratch_shapes=[
            pltpu.VMEM((Q, NN), jnp.float32),
            pltpu.VMEM((Q, NN), jnp.int32),
        ],
    )(inputs, tt_pad)


NC = 2
NS = 16
NW = NC * NS
QPW = Q // NW


def _gather_mean(target_data, idx_flat):
    mesh = plsc.VectorSubcoreMesh(
        core_axis_name="c", subcore_axis_name="s", num_cores=NC,
        num_subcores=NS)

    @functools.partial(
        pl.kernel,
        out_type=jax.ShapeDtypeStruct((Q, D), jnp.float32),
        mesh=mesh,
        scratch_types=[
            pltpu.VMEM((QPW * NN,), jnp.int32),
            pltpu.VMEM((NN, D), jnp.float32),
            pltpu.VMEM((QPW, D), jnp.float32),
            pltpu.SemaphoreType.DMA,
        ],
    )
    def sc_kernel(table_hbm, idx_hbm, out_hbm, idx_v, rows_v, ostage_v, sem):
        wid = lax.axis_index("s") * NC + lax.axis_index("c")
        qbase = wid * QPW
        pltpu.sync_copy(idx_hbm.at[pl.ds(qbase * NN, QPW * NN)], idx_v)

        def per_query(qi, carry):
            off = pl.multiple_of(qi * NN, 8)
            pltpu.async_copy(
                table_hbm.at[idx_v.at[pl.ds(off, NN)]], rows_v, sem).wait()
            for c in range(D // 16):
                def body(r, acc):
                    return acc + rows_v[r, pl.ds(c * 16, 16)]
                acc = lax.fori_loop(0, NN, body, jnp.zeros((16,), jnp.float32))
                ostage_v[qi, pl.ds(c * 16, 16)] = acc * (1.0 / NN)
            return carry

        lax.fori_loop(0, QPW, per_query, 0)
        pltpu.sync_copy(ostage_v, out_hbm.at[pl.ds(qbase, QPW)])

    return sc_kernel(target_data, idx_flat)


def kernel(inputs, target_data):
    tt_pad = jnp.pad(target_data, ((0, K_PAD - K), (0, 0))).T
    idx = _topk_indices(inputs, tt_pad)
    return _gather_mean(target_data, idx.reshape(-1))


# speedup vs baseline: 4.0889x; 2.8767x over previous
import functools

import jax
import jax.numpy as jnp
from jax import lax
from jax.experimental import pallas as pl
from jax.experimental.pallas import tpu as pltpu
from jax.experimental.pallas import tpu_sc as plsc

Q = 1024
D = 128
K = 100000
NN = 32

BK = 2048
NKT = (K + BK - 1) // BK
K_PAD = NKT * BK

_BIG = 3.0e38
_BIGI = 2**31 - 1


def _topk_kernel(q_ref, tt_ref, oidx_ref, rv_ref, ri_ref, s_ref):
    kt = pl.program_id(0)

    @pl.when(kt == 0)
    def _init():
        rv_ref[...] = jnp.full((Q, NN), _BIG, jnp.float32)
        ri_ref[...] = jnp.full((Q, NN), _BIGI, jnp.int32)

    qm = q_ref[...]
    tt = tt_ref[...]
    t2 = jnp.sum(tt * tt, axis=0, keepdims=True)
    s = t2 - 2.0 * jnp.dot(qm, tt, preferred_element_type=jnp.float32)

    col = lax.broadcasted_iota(jnp.int32, (Q, BK), 1)
    gcol = col + kt * BK
    s = jnp.where(gcol >= K, _BIG, s)
    s_ref[...] = s

    tau = rv_ref[:, NN - 1:NN]
    itau = ri_ref[:, NN - 1:NN]
    beats = (s < tau) | ((s == tau) & (gcol < itau))
    cnt = jnp.sum(beats.astype(jnp.int32), axis=1, keepdims=True)
    need = jnp.max(jnp.minimum(cnt, NN))

    lane = lax.broadcasted_iota(jnp.int32, (Q, NN), 1)

    def body(j, carry):
        sv = s_ref[...]
        m = jnp.min(sv, axis=1, keepdims=True)
        sel = jnp.min(jnp.where(sv == m, col, _BIGI), axis=1, keepdims=True)
        s_ref[...] = jnp.where(col == sel, _BIG, sv)
        gsel = sel + kt * BK
        rv = rv_ref[...]
        ri = ri_ref[...]
        before = (rv < m) | ((rv == m) & (ri < gsel))
        pos = jnp.sum(before.astype(jnp.int32), axis=1, keepdims=True)
        rvs = pltpu.roll(rv, 1, axis=1)
        ris = pltpu.roll(ri, 1, axis=1)
        rv_ref[...] = jnp.where(lane < pos, rv,
                                jnp.where(lane == pos, m, rvs))
        ri_ref[...] = jnp.where(lane < pos, ri,
                                jnp.where(lane == pos, gsel, ris))
        return carry

    lax.fori_loop(0, need, body, 0)

    @pl.when(kt == NKT - 1)
    def _out():
        oidx_ref[...] = ri_ref[...]


def _topk_indices(inputs, tt_pad):
    return pl.pallas_call(
        _topk_kernel,
        grid=(NKT,),
        in_specs=[
            pl.BlockSpec((Q, D), lambda k: (0, 0)),
            pl.BlockSpec((D, BK), lambda k: (0, k)),
        ],
        out_specs=pl.BlockSpec((Q, NN), lambda k: (0, 0)),
        out_shape=jax.ShapeDtypeStruct((Q, NN), jnp.int32),
        scratch_shapes=[
            pltpu.VMEM((Q, NN), jnp.float32),
            pltpu.VMEM((Q, NN), jnp.int32),
            pltpu.VMEM((Q, BK), jnp.float32),
        ],
    )(inputs, tt_pad)


NC = 2
NS = 16
NW = NC * NS
QPW = Q // NW


def _gather_mean(target_data, idx_flat):
    mesh = plsc.VectorSubcoreMesh(
        core_axis_name="c", subcore_axis_name="s", num_cores=NC,
        num_subcores=NS)

    @functools.partial(
        pl.kernel,
        out_type=jax.ShapeDtypeStruct((Q, D), jnp.float32),
        mesh=mesh,
        scratch_types=[
            pltpu.VMEM((QPW * NN,), jnp.int32),
            pltpu.VMEM((NN, D), jnp.float32),
            pltpu.VMEM((QPW, D), jnp.float32),
            pltpu.SemaphoreType.DMA,
        ],
    )
    def sc_kernel(table_hbm, idx_hbm, out_hbm, idx_v, rows_v, ostage_v, sem):
        wid = lax.axis_index("s") * NC + lax.axis_index("c")
        qbase = wid * QPW
        pltpu.sync_copy(idx_hbm.at[pl.ds(qbase * NN, QPW * NN)], idx_v)

        def per_query(qi, carry):
            off = pl.multiple_of(qi * NN, 8)
            pltpu.async_copy(
                table_hbm.at[idx_v.at[pl.ds(off, NN)]], rows_v, sem).wait()
            for c in range(D // 16):
                def body(r, acc):
                    return acc + rows_v[r, pl.ds(c * 16, 16)]
                acc = lax.fori_loop(0, NN, body, jnp.zeros((16,), jnp.float32))
                ostage_v[qi, pl.ds(c * 16, 16)] = acc * (1.0 / NN)
            return carry

        lax.fori_loop(0, QPW, per_query, 0)
        pltpu.sync_copy(ostage_v, out_hbm.at[pl.ds(qbase, QPW)])

    return sc_kernel(target_data, idx_flat)


def kernel(inputs, target_data):
    tt_pad = jnp.pad(target_data, ((0, K_PAD - K), (0, 0))).T
    idx = _topk_indices(inputs, tt_pad)
    return _gather_mean(target_data, idx.reshape(-1))
